# baseline (speedup 1.0000x reference)
import functools

import jax
import jax.numpy as jnp
from jax import lax
from jax.experimental import pallas as pl
from jax.experimental.pallas import tpu as pltpu
from jax.experimental.pallas import tpu_sc as plsc

B = 16384
D = 128
K = 20
R = K + 1
L = 16
NC = 2
NS = 16
NW = NC * NS
BPW = B // NW
S = 8
NSUB = BPW // S
NROWS = S * K
NSPLIT = (128, 32)
NBUF = 4


def _sc_scores(tid, cid, nid, ttab, ctab):
    mesh = plsc.VectorSubcoreMesh(core_axis_name="c", subcore_axis_name="s")

    @functools.partial(
        pl.kernel,
        out_type=jax.ShapeDtypeStruct((B * R,), jnp.float32),
        mesh=mesh,
        scratch_types=[
            pltpu.VMEM((BPW,), jnp.int32),
            pltpu.VMEM((BPW,), jnp.int32),
            pltpu.VMEM((BPW * K,), jnp.int32),
            pltpu.VMEM((NBUF, S, D), jnp.float32),
            pltpu.VMEM((NBUF, S, D), jnp.float32),
            pltpu.VMEM((NBUF, NROWS, D), jnp.float32),
        ] + [pltpu.VMEM((S * R + L,), jnp.float32)] * NBUF
          + [pltpu.SemaphoreType.DMA] * (2 * NBUF),
        compiler_params=pltpu.CompilerParams(needs_layout_passes=False),
    )
    def k(tid_h, cid_h, nid_h, ttab_h, ctab_h, out_h,
          tidx, cidx, nidx, trows, crows, nrows, *svsem):
        wid = lax.axis_index("s") * NC + lax.axis_index("c")
        b0w = wid * BPW
        pltpu.sync_copy(tid_h.at[pl.ds(b0w, BPW)], tidx)
        pltpu.sync_copy(cid_h.at[pl.ds(b0w, BPW)], cidx)
        pltpu.sync_copy(nid_h.at[pl.ds(b0w * K, BPW * K)], nidx)
        svs = svsem[:NBUF]
        gsem = svsem[NBUF:2 * NBUF]
        osem = svsem[2 * NBUF:]
        lastlane = lax.iota(jnp.int32, L) == (L - 1)

        def mk_gather(si, p):
            cps = [
                pltpu.make_async_copy(
                    ttab_h.at[tidx.at[pl.ds(si * S, S)]],
                    trows.at[p], gsem[p]),
                pltpu.make_async_copy(
                    ctab_h.at[cidx.at[pl.ds(si * S, S)]],
                    crows.at[p], gsem[p]),
            ]
            off = 0
            for w in NSPLIT:
                cps.append(pltpu.make_async_copy(
                    ctab_h.at[nidx.at[pl.ds(si * NROWS + off, w)]],
                    nrows.at[p, pl.ds(off, w)], gsem[p]))
                off += w
            return cps

        def mk_out(si, p):
            return pltpu.make_async_copy(
                svs[p].at[pl.ds(0, S * R)],
                out_h.at[pl.ds((b0w + si * S) * R, S * R)], osem[p])

        for pp in range(NBUF - 1):
            for c in mk_gather(pp, pp):
                c.start()

        def outer(oi, carry):
            for p in range(NBUF):
                si = oi * NBUF + p

                @pl.when(si + NBUF - 1 < NSUB)
                def _prefetch():
                    for c in mk_gather(si + NBUF - 1, (p + NBUF - 1) % NBUF):
                        c.start()

                @pl.when(oi > 0)
                def _drain_out():
                    mk_out(si - NBUF, p).wait()

                for c in mk_gather(si, p):
                    c.wait()

                @plsc.parallel_loop(0, S)
                def per_b(b):
                    t = [trows[p, b, pl.ds(j * L, L)] for j in range(8)]

                    def dot_store(src_row_ref, out_pos):
                        q = [t[j] * src_row_ref[pl.ds(j * L, L)]
                             for j in range(8)]
                        q = [q[2 * j] + q[2 * j + 1] for j in range(4)]
                        acc = (q[0] + q[1]) + (q[2] + q[3])
                        cs = plsc.cumsum(acc)
                        plsc.store_compressed(
                            svs[p].at[pl.ds(out_pos, L)], cs, mask=lastlane)

                    dot_store(crows.at[p, b], b * R)
                    for r in range(K):
                        dot_store(nrows.at[p, b * K + r], b * R + 1 + r)

                mk_out(si, p).start()
            return carry

        lax.fori_loop(0, NSUB // NBUF, outer, 0)
        for pp in range(NBUF):
            mk_out(NSUB - NBUF + pp, pp).wait()

    return k(tid, cid, nid, ttab, ctab)


RB = B * R // D


def _tc_loss(scores):
    sr = scores.reshape(RB, D)

    def body(s_ref, o_ref):
        row = lax.broadcasted_iota(jnp.int32, (RB, D), 0)
        col = lax.broadcasted_iota(jnp.int32, (RB, D), 1)
        ispos = ((row * D + col) % R) == 0
        s = s_ref[...]
        x = jnp.where(ispos, s, -s)
        ls = jnp.minimum(x, 0.0) - jnp.log1p(jnp.exp(-jnp.abs(x)))
        o_ref[0, 0] = -jnp.sum(ls) / B

    out = pl.pallas_call(
        body,
        out_shape=jax.ShapeDtypeStruct((1, 1), jnp.float32),
        out_specs=pl.BlockSpec(memory_space=pltpu.SMEM),
    )(sr)
    return out[0, 0]


def kernel(target_ids, context_ids, neg_ids, target_table, context_table):
    tid = target_ids.astype(jnp.int32)
    cid = context_ids.astype(jnp.int32)
    nid = neg_ids.astype(jnp.int32).reshape(B * K)
    scores = _sc_scores(tid, cid, nid, target_table, context_table)
    return _tc_loss(scores)


# speedup vs baseline: 1.4156x; 1.4156x over previous
import functools

import jax
import jax.numpy as jnp
from jax import lax
from jax.experimental import pallas as pl
from jax.experimental.pallas import tpu as pltpu
from jax.experimental.pallas import tpu_sc as plsc

B = 16384
D = 128
K = 20
R = K + 1
L = 16
NC = 2
NS = 16
NW = NC * NS
BPW = B // NW
S = 16
NSUB = BPW // S
NROWS = S * K
NSPLIT = (128, 128, 64)


def _sc_scores(tid, cid, nid, ttab, ctab):
    mesh = plsc.VectorSubcoreMesh(core_axis_name="c", subcore_axis_name="s")

    @functools.partial(
        pl.kernel,
        out_type=jax.ShapeDtypeStruct((B * R,), jnp.float32),
        mesh=mesh,
        scratch_types=[
            pltpu.VMEM((BPW,), jnp.int32),
            pltpu.VMEM((BPW,), jnp.int32),
            pltpu.VMEM((BPW * K,), jnp.int32),
            pltpu.VMEM((2, S, D), jnp.float32),
            pltpu.VMEM((2, S, D), jnp.float32),
            pltpu.VMEM((2, NROWS, D), jnp.float32),
            pltpu.VMEM((S * R + L,), jnp.float32),
            pltpu.VMEM((S * R + L,), jnp.float32),
            pltpu.SemaphoreType.DMA,
            pltpu.SemaphoreType.DMA,
            pltpu.SemaphoreType.DMA,
            pltpu.SemaphoreType.DMA,
        ],
        compiler_params=pltpu.CompilerParams(needs_layout_passes=False),
    )
    def k(tid_h, cid_h, nid_h, ttab_h, ctab_h, out_h,
          tidx, cidx, nidx, trows, crows, nrows, sv0, sv1, g0, g1, o0, o1):
        wid = lax.axis_index("s") * NC + lax.axis_index("c")
        b0w = wid * BPW
        pltpu.sync_copy(tid_h.at[pl.ds(b0w, BPW)], tidx)
        pltpu.sync_copy(cid_h.at[pl.ds(b0w, BPW)], cidx)
        pltpu.sync_copy(nid_h.at[pl.ds(b0w * K, BPW * K)], nidx)
        svs = (sv0, sv1)
        gsem = (g0, g1)
        osem = (o0, o1)
        lastlane = lax.iota(jnp.int32, L) == (L - 1)

        def mk_gather(si, p):
            cps = [
                pltpu.make_async_copy(
                    ttab_h.at[tidx.at[pl.ds(si * S, S)]],
                    trows.at[p], gsem[p]),
                pltpu.make_async_copy(
                    ctab_h.at[cidx.at[pl.ds(si * S, S)]],
                    crows.at[p], gsem[p]),
            ]
            off = 0
            for w in NSPLIT:
                cps.append(pltpu.make_async_copy(
                    ctab_h.at[nidx.at[pl.ds(si * NROWS + off, w)]],
                    nrows.at[p, pl.ds(off, w)], gsem[p]))
                off += w
            return cps

        def mk_out(si, p):
            return pltpu.make_async_copy(
                svs[p].at[pl.ds(0, S * R)],
                out_h.at[pl.ds((b0w + si * S) * R, S * R)], osem[p])

        for c in mk_gather(0, 0):
            c.start()

        def outer(oi, carry):
            for p in range(2):
                si = oi * 2 + p

                @pl.when(si + 1 < NSUB)
                def _prefetch():
                    for c in mk_gather(si + 1, 1 - p):
                        c.start()

                @pl.when(oi > 0)
                def _drain_out():
                    mk_out(si - 2, p).wait()

                for c in mk_gather(si, p):
                    c.wait()

                @plsc.parallel_loop(0, S)
                def per_b(b):
                    t = [trows[p, b, pl.ds(j * L, L)] for j in range(8)]
                    tn = [-v for v in t]

                    def dot_store(tv, src_row_ref, out_pos):
                        q = [tv[j] * src_row_ref[pl.ds(j * L, L)]
                             for j in range(8)]
                        q = [q[2 * j] + q[2 * j + 1] for j in range(4)]
                        acc = (q[0] + q[1]) + (q[2] + q[3])
                        cs = plsc.cumsum(acc)
                        plsc.store_compressed(
                            svs[p].at[pl.ds(out_pos, L)], cs, mask=lastlane)

                    dot_store(t, crows.at[p, b], b * R)
                    for r in range(K):
                        dot_store(tn, nrows.at[p, b * K + r],
                                  b * R + 1 + r)

                mk_out(si, p).start()
            return carry

        lax.fori_loop(0, NSUB // 2, outer, 0)
        mk_out(NSUB - 2, 0).wait()
        mk_out(NSUB - 1, 1).wait()

    return k(tid, cid, nid, ttab, ctab)


RB = B * R // D


def _tc_loss(scores):
    sr = scores.reshape(RB, D)

    def body(s_ref, o_ref):
        x = s_ref[...]
        ls = jnp.minimum(x, 0.0) - jnp.log1p(jnp.exp(-jnp.abs(x)))
        o_ref[0, 0] = -jnp.sum(ls) / B

    out = pl.pallas_call(
        body,
        out_shape=jax.ShapeDtypeStruct((1, 1), jnp.float32),
        out_specs=pl.BlockSpec(memory_space=pltpu.SMEM),
    )(sr)
    return out[0, 0]


def kernel(target_ids, context_ids, neg_ids, target_table, context_table):
    tid = target_ids.astype(jnp.int32)
    cid = context_ids.astype(jnp.int32)
    nid = neg_ids.astype(jnp.int32).reshape(B * K)
    scores = _sc_scores(tid, cid, nid, target_table, context_table)
    return _tc_loss(scores)


# speedup vs baseline: 1.4488x; 1.0235x over previous
import functools

import jax
import jax.numpy as jnp
from jax import lax
from jax.experimental import pallas as pl
from jax.experimental.pallas import tpu as pltpu
from jax.experimental.pallas import tpu_sc as plsc

B = 16384
D = 128
K = 20
R = K + 1
L = 16
NC = 2
NS = 16
NW = NC * NS
BPW = B // NW
S = 16
NSUB = BPW // S
NROWS = S * K


def _sc_scores(tid, cid, nid, ttab, ctab):
    mesh = plsc.VectorSubcoreMesh(core_axis_name="c", subcore_axis_name="s")

    @functools.partial(
        pl.kernel,
        out_type=jax.ShapeDtypeStruct((B * R,), jnp.float32),
        mesh=mesh,
        scratch_types=[
            pltpu.VMEM((BPW,), jnp.int32),
            pltpu.VMEM((BPW,), jnp.int32),
            pltpu.VMEM((S, K), jnp.int32),
            pltpu.VMEM((S, K), jnp.int32),
            pltpu.VMEM((2, S, D), jnp.float32),
            pltpu.VMEM((2, S, D), jnp.float32),
            pltpu.VMEM((2, NROWS, D), jnp.float32),
            pltpu.VMEM((S * R + L,), jnp.float32),
            pltpu.VMEM((S * R + L,), jnp.float32),
            pltpu.SemaphoreType.DMA,
            pltpu.SemaphoreType.DMA,
            pltpu.SemaphoreType.DMA,
            pltpu.SemaphoreType.DMA,
            pltpu.SemaphoreType.DMA,
            pltpu.SemaphoreType.DMA,
        ],
        compiler_params=pltpu.CompilerParams(needs_layout_passes=False),
    )
    def k(tid_h, cid_h, nid_h, ttab_h, ctab_h, out_h,
          tidx, cidx, ni0, ni1, trows, crows, nrows, sv0, sv1,
          g0, g1, o0, o1, i0, i1):
        wid = lax.axis_index("s") * NC + lax.axis_index("c")
        b0w = wid * BPW
        pltpu.sync_copy(tid_h.at[pl.ds(b0w, BPW)], tidx)
        pltpu.sync_copy(cid_h.at[pl.ds(b0w, BPW)], cidx)
        svs = (sv0, sv1)
        nis = (ni0, ni1)
        gsem = (g0, g1)
        osem = (o0, o1)
        isem = (i0, i1)
        lastlane = lax.iota(jnp.int32, L) == (L - 1)

        def mk_idx(si, p):
            return pltpu.make_async_copy(
                nid_h.at[pl.ds(b0w + si * S, S)], nis[p], isem[p])

        def mk_gather(si, p):
            cps = [
                pltpu.make_async_copy(
                    ttab_h.at[tidx.at[pl.ds(si * S, S)]],
                    trows.at[p], gsem[p]),
                pltpu.make_async_copy(
                    ctab_h.at[cidx.at[pl.ds(si * S, S)]],
                    crows.at[p], gsem[p]),
            ]
            cps += [
                pltpu.make_async_copy(
                    ctab_h.at[nis[p].at[bb]],
                    nrows.at[p, pl.ds(bb * K, K)], gsem[p])
                for bb in range(S)
            ]
            return cps

        def mk_out(si, p):
            return pltpu.make_async_copy(
                svs[p].at[pl.ds(0, S * R)],
                out_h.at[pl.ds((b0w + si * S) * R, S * R)], osem[p])

        pltpu.sync_copy(nid_h.at[pl.ds(b0w, S)], ni0)
        pltpu.sync_copy(nid_h.at[pl.ds(b0w + S, S)], ni1)
        for c in mk_gather(0, 0):
            c.start()

        def outer(oi, carry):
            for p in range(2):
                si = oi * 2 + p

                @pl.when(jnp.logical_and(si + 1 < NSUB, si >= 1))
                def _wait_idx():
                    mk_idx(si + 1, 1 - p).wait()

                @pl.when(si + 1 < NSUB)
                def _prefetch():
                    for c in mk_gather(si + 1, 1 - p):
                        c.start()

                for c in mk_gather(si, p):
                    c.wait()

                @pl.when(si + 2 < NSUB)
                def _stage_idx():
                    mk_idx(si + 2, p).start()

                @pl.when(oi > 0)
                def _drain_out():
                    mk_out(si - 2, p).wait()

                @plsc.parallel_loop(0, S)
                def per_b(b):
                    t = [trows[p, b, pl.ds(j * L, L)] for j in range(8)]
                    tn = [-v for v in t]

                    def dot_store(tv, src_row_ref, out_pos):
                        q = [tv[j] * src_row_ref[pl.ds(j * L, L)]
                             for j in range(8)]
                        q = [q[2 * j] + q[2 * j + 1] for j in range(4)]
                        acc = (q[0] + q[1]) + (q[2] + q[3])
                        cs = plsc.cumsum(acc)
                        plsc.store_compressed(
                            svs[p].at[pl.ds(out_pos, L)], cs, mask=lastlane)

                    dot_store(t, crows.at[p, b], b * R)
                    for r in range(K):
                        dot_store(tn, nrows.at[p, b * K + r],
                                  b * R + 1 + r)

                mk_out(si, p).start()
            return carry

        lax.fori_loop(0, NSUB // 2, outer, 0)
        mk_out(NSUB - 2, 0).wait()
        mk_out(NSUB - 1, 1).wait()

    return k(tid, cid, nid, ttab, ctab)


RB = B * R // D


def _tc_loss(scores):
    sr = scores.reshape(RB, D)

    def body(s_ref, o_ref):
        x = s_ref[...]
        ls = jnp.minimum(x, 0.0) - jnp.log1p(jnp.exp(-jnp.abs(x)))
        o_ref[0, 0] = -jnp.sum(ls) / B

    out = pl.pallas_call(
        body,
        out_shape=jax.ShapeDtypeStruct((1, 1), jnp.float32),
        out_specs=pl.BlockSpec(memory_space=pltpu.SMEM),
    )(sr)
    return out[0, 0]


def kernel(target_ids, context_ids, neg_ids, target_table, context_table):
    tid = target_ids.astype(jnp.int32)
    cid = context_ids.astype(jnp.int32)
    nid = neg_ids.astype(jnp.int32)
    scores = _sc_scores(tid, cid, nid, target_table, context_table)
    return _tc_loss(scores)


# speedup vs baseline: 1.4493x; 1.0003x over previous
import functools

import jax
import jax.numpy as jnp
from jax import lax
from jax.experimental import pallas as pl
from jax.experimental.pallas import tpu as pltpu
from jax.experimental.pallas import tpu_sc as plsc

B = 16384
D = 128
K = 20
R = K + 1
L = 16
NC = 2
NS = 16
NW = NC * NS
BPW = B // NW
S = 16
NSUB = BPW // S
NROWS = S * K


def _sc_scores(tid, cid, nid, ttab, ctab):
    mesh = plsc.VectorSubcoreMesh(core_axis_name="c", subcore_axis_name="s")

    @functools.partial(
        pl.kernel,
        out_type=jax.ShapeDtypeStruct((B * R,), jnp.float32),
        mesh=mesh,
        scratch_types=[
            pltpu.VMEM((BPW,), jnp.int32),
            pltpu.VMEM((BPW,), jnp.int32),
            pltpu.VMEM((S, K), jnp.int32),
            pltpu.VMEM((S, K), jnp.int32),
            pltpu.VMEM((2, S, D), jnp.float32),
            pltpu.VMEM((2, S, D), jnp.float32),
            pltpu.VMEM((2, NROWS, D), jnp.float32),
            pltpu.VMEM((S * R + L,), jnp.float32),
            pltpu.VMEM((S * R + L,), jnp.float32),
            pltpu.SemaphoreType.DMA,
            pltpu.SemaphoreType.DMA,
            pltpu.SemaphoreType.DMA,
            pltpu.SemaphoreType.DMA,
            pltpu.SemaphoreType.DMA,
            pltpu.SemaphoreType.DMA,
        ],
        compiler_params=pltpu.CompilerParams(needs_layout_passes=False, use_tc_tiling_on_sc=True),
    )
    def k(tid_h, cid_h, nid_h, ttab_h, ctab_h, out_h,
          tidx, cidx, ni0, ni1, trows, crows, nrows, sv0, sv1,
          g0, g1, o0, o1, i0, i1):
        wid = lax.axis_index("s") * NC + lax.axis_index("c")
        b0w = wid * BPW
        pltpu.sync_copy(tid_h.at[pl.ds(b0w, BPW)], tidx)
        pltpu.sync_copy(cid_h.at[pl.ds(b0w, BPW)], cidx)
        svs = (sv0, sv1)
        nis = (ni0, ni1)
        gsem = (g0, g1)
        osem = (o0, o1)
        isem = (i0, i1)
        lastlane = lax.iota(jnp.int32, L) == (L - 1)

        def mk_idx(si, p):
            return pltpu.make_async_copy(
                nid_h.at[pl.ds(b0w + si * S, S)], nis[p], isem[p])

        def mk_gather(si, p):
            cps = [
                pltpu.make_async_copy(
                    ttab_h.at[tidx.at[pl.ds(si * S, S)]],
                    trows.at[p], gsem[p]),
                pltpu.make_async_copy(
                    ctab_h.at[cidx.at[pl.ds(si * S, S)]],
                    crows.at[p], gsem[p]),
            ]
            cps += [
                pltpu.make_async_copy(
                    ctab_h.at[nis[p].at[bb]],
                    nrows.at[p, pl.ds(bb * K, K)], gsem[p])
                for bb in range(S)
            ]
            return cps

        def mk_out(si, p):
            return pltpu.make_async_copy(
                svs[p].at[pl.ds(0, S * R)],
                out_h.at[pl.ds((b0w + si * S) * R, S * R)], osem[p])

        pltpu.sync_copy(nid_h.at[pl.ds(b0w, S)], ni0)
        pltpu.sync_copy(nid_h.at[pl.ds(b0w + S, S)], ni1)
        for c in mk_gather(0, 0):
            c.start()

        def outer(oi, carry):
            for p in range(2):
                si = oi * 2 + p

                @pl.when(jnp.logical_and(si + 1 < NSUB, si >= 1))
                def _wait_idx():
                    mk_idx(si + 1, 1 - p).wait()

                @pl.when(si + 1 < NSUB)
                def _prefetch():
                    for c in mk_gather(si + 1, 1 - p):
                        c.start()

                for c in mk_gather(si, p):
                    c.wait()

                @pl.when(si + 2 < NSUB)
                def _stage_idx():
                    mk_idx(si + 2, p).start()

                @pl.when(oi > 0)
                def _drain_out():
                    mk_out(si - 2, p).wait()

                @plsc.parallel_loop(0, S)
                def per_b(b):
                    t = [trows[p, b, pl.ds(j * L, L)] for j in range(8)]
                    tn = [-v for v in t]

                    def dot_store(tv, src_row_ref, out_pos):
                        q = [tv[j] * src_row_ref[pl.ds(j * L, L)]
                             for j in range(8)]
                        q = [q[2 * j] + q[2 * j + 1] for j in range(4)]
                        acc = (q[0] + q[1]) + (q[2] + q[3])
                        cs = plsc.cumsum(acc)
                        plsc.store_compressed(
                            svs[p].at[pl.ds(out_pos, L)], cs, mask=lastlane)

                    dot_store(t, crows.at[p, b], b * R)
                    for r in range(K):
                        dot_store(tn, nrows.at[p, b * K + r],
                                  b * R + 1 + r)

                mk_out(si, p).start()
            return carry

        lax.fori_loop(0, NSUB // 2, outer, 0)
        mk_out(NSUB - 2, 0).wait()
        mk_out(NSUB - 1, 1).wait()

    return k(tid, cid, nid, ttab, ctab)


RB = B * R // D


def _tc_loss(scores):
    sr = scores.reshape(RB, D)

    def body(s_ref, o_ref):
        x = s_ref[...]
        ls = jnp.minimum(x, 0.0) - jnp.log1p(jnp.exp(-jnp.abs(x)))
        o_ref[0, 0] = -jnp.sum(ls) / B

    out = pl.pallas_call(
        body,
        out_shape=jax.ShapeDtypeStruct((1, 1), jnp.float32),
        out_specs=pl.BlockSpec(memory_space=pltpu.SMEM),
    )(sr)
    return out[0, 0]


def kernel(target_ids, context_ids, neg_ids, target_table, context_table):
    tid = target_ids.astype(jnp.int32)
    cid = context_ids.astype(jnp.int32)
    nid = neg_ids.astype(jnp.int32)
    scores = _sc_scores(tid, cid, nid, target_table, context_table)
    return _tc_loss(scores)


# speedup vs baseline: 1.4869x; 1.0260x over previous
import functools

import jax
import jax.numpy as jnp
from jax import lax
from jax.experimental import pallas as pl
from jax.experimental.pallas import tpu as pltpu
from jax.experimental.pallas import tpu_sc as plsc

B = 16384
D = 128
K = 20
R = K + 1
L = 16
NC = 2
NS = 16
NW = NC * NS
BPW = B // NW
S = 16
NSUB = BPW // S
NROWS = S * K
NSPLIT = (128, 128, 64)


def _sc_scores(tid, cid, nid, ttab, ctab):
    mesh = plsc.VectorSubcoreMesh(core_axis_name="c", subcore_axis_name="s")

    @functools.partial(
        pl.kernel,
        out_type=jax.ShapeDtypeStruct((B * R,), jnp.float32),
        mesh=mesh,
        scratch_types=[
            pltpu.VMEM((BPW,), jnp.int32),
            pltpu.VMEM((BPW,), jnp.int32),
            pltpu.VMEM((S, K), jnp.int32),
            pltpu.VMEM((S, K), jnp.int32),
            pltpu.VMEM((2, S, D), jnp.float32),
            pltpu.VMEM((2, S, D), jnp.float32),
            pltpu.VMEM((2, NROWS, D), jnp.float32),
            pltpu.VMEM((S * R + L,), jnp.float32),
            pltpu.VMEM((S * R + L,), jnp.float32),
            pltpu.VMEM((NROWS,), jnp.int32),
            pltpu.VMEM((NROWS,), jnp.int32),
            pltpu.SemaphoreType.DMA,
            pltpu.SemaphoreType.DMA,
            pltpu.SemaphoreType.DMA,
            pltpu.SemaphoreType.DMA,
            pltpu.SemaphoreType.DMA,
            pltpu.SemaphoreType.DMA,
        ],
        compiler_params=pltpu.CompilerParams(needs_layout_passes=False),
    )
    def k(tid_h, cid_h, nid_h, ttab_h, ctab_h, out_h,
          tidx, cidx, ni0, ni1, trows, crows, nrows, sv0, sv1,
          fl0, fl1, g0, g1, o0, o1, i0, i1):
        wid = lax.axis_index("s") * NC + lax.axis_index("c")
        b0w = wid * BPW
        pltpu.sync_copy(tid_h.at[pl.ds(b0w, BPW)], tidx)
        pltpu.sync_copy(cid_h.at[pl.ds(b0w, BPW)], cidx)
        svs = (sv0, sv1)
        nis = (ni0, ni1)
        fls = (fl0, fl1)
        gsem = (g0, g1)
        osem = (o0, o1)
        isem = (i0, i1)
        lastlane = lax.iota(jnp.int32, L) == (L - 1)

        def mk_idx(si, p):
            return pltpu.make_async_copy(
                nid_h.at[pl.ds(b0w + si * S, S)], nis[p], isem[p])

        def mk_gather(si, p):
            cps = [
                pltpu.make_async_copy(
                    ttab_h.at[tidx.at[pl.ds(si * S, S)]],
                    trows.at[p], gsem[p]),
                pltpu.make_async_copy(
                    ctab_h.at[cidx.at[pl.ds(si * S, S)]],
                    crows.at[p], gsem[p]),
            ]
            off = 0
            for w in NSPLIT:
                cps.append(pltpu.make_async_copy(
                    ctab_h.at[fls[p].at[pl.ds(off, w)]],
                    nrows.at[p, pl.ds(off, w)], gsem[p]))
                off += w
            return cps

        def mk_out(si, p):
            return pltpu.make_async_copy(
                svs[p].at[pl.ds(0, S * R)],
                out_h.at[pl.ds((b0w + si * S) * R, S * R)], osem[p])

        def build_flat(q):
            for bb in range(S):
                fls[q][pl.ds(bb * K, L)] = nis[q][bb, pl.ds(0, L)]
                fls[q][pl.ds(bb * K + 4, L)] = nis[q][bb, pl.ds(4, L)]

        pltpu.sync_copy(nid_h.at[pl.ds(b0w, S)], ni0)
        pltpu.sync_copy(nid_h.at[pl.ds(b0w + S, S)], ni1)
        build_flat(0)
        for c in mk_gather(0, 0):
            c.start()

        def outer(oi, carry):
            for p in range(2):
                si = oi * 2 + p

                @pl.when(jnp.logical_and(si + 1 < NSUB, si >= 1))
                def _wait_idx():
                    mk_idx(si + 1, 1 - p).wait()

                @pl.when(si + 1 < NSUB)
                def _prefetch():
                    build_flat(1 - p)
                    for c in mk_gather(si + 1, 1 - p):
                        c.start()

                for c in mk_gather(si, p):
                    c.wait()

                @pl.when(si + 2 < NSUB)
                def _stage_idx():
                    mk_idx(si + 2, p).start()

                @pl.when(oi > 0)
                def _drain_out():
                    mk_out(si - 2, p).wait()

                @plsc.parallel_loop(0, S)
                def per_b(b):
                    t = [trows[p, b, pl.ds(j * L, L)] for j in range(8)]
                    tn = [-v for v in t]

                    def dot_store(tv, src_row_ref, out_pos):
                        q = [tv[j] * src_row_ref[pl.ds(j * L, L)]
                             for j in range(8)]
                        q = [q[2 * j] + q[2 * j + 1] for j in range(4)]
                        acc = (q[0] + q[1]) + (q[2] + q[3])
                        cs = plsc.cumsum(acc)
                        plsc.store_compressed(
                            svs[p].at[pl.ds(out_pos, L)], cs, mask=lastlane)

                    dot_store(t, crows.at[p, b], b * R)
                    for r in range(K):
                        dot_store(tn, nrows.at[p, b * K + r],
                                  b * R + 1 + r)

                mk_out(si, p).start()
            return carry

        lax.fori_loop(0, NSUB // 2, outer, 0)
        mk_out(NSUB - 2, 0).wait()
        mk_out(NSUB - 1, 1).wait()

    return k(tid, cid, nid, ttab, ctab)


RB = B * R // D


def _tc_loss(scores):
    sr = scores.reshape(RB, D)

    def body(s_ref, o_ref):
        x = s_ref[...]
        ls = jnp.minimum(x, 0.0) - jnp.log1p(jnp.exp(-jnp.abs(x)))
        o_ref[0, 0] = -jnp.sum(ls) / B

    out = pl.pallas_call(
        body,
        out_shape=jax.ShapeDtypeStruct((1, 1), jnp.float32),
        out_specs=pl.BlockSpec(memory_space=pltpu.SMEM),
    )(sr)
    return out[0, 0]


def kernel(target_ids, context_ids, neg_ids, target_table, context_table):
    tid = target_ids.astype(jnp.int32)
    cid = context_ids.astype(jnp.int32)
    nid = neg_ids.astype(jnp.int32)
    scores = _sc_scores(tid, cid, nid, target_table, context_table)
    return _tc_loss(scores)
